# Initial kernel scaffold; baseline (speedup 1.0000x reference)
import jax
import jax.numpy as jnp
from jax.experimental import pallas as pl


def kernel(tokens, table):
    raise NotImplementedError("write your pallas kernel here")


# speedup vs baseline: 1.3627x; 1.3627x over previous
import functools

import jax
import jax.numpy as jnp
from jax import lax
from jax.experimental import pallas as pl
from jax.experimental.pallas import tpu as pltpu
from jax.experimental.pallas import tpu_sc as plsc

B = 1024
MAXLEN = 1024
EMB = 128
VOCAB = 21

NC = 2
NS = 16
NW = NC * NS
NTOK = B * MAXLEN
TOK_PER_W = NTOK // NW
CH = 128
NCH = TOK_PER_W // CH

@functools.cache
def _make_emb_sc():
    mesh = plsc.VectorSubcoreMesh(
        core_axis_name="c", subcore_axis_name="s", num_cores=NC, num_subcores=NS
    )
    return functools.partial(
        pl.kernel,
        out_type=jax.ShapeDtypeStruct((NTOK, EMB), jnp.float32),
        mesh=mesh,
        scratch_types=[
            pltpu.VMEM((NCH, CH), jnp.int32),
            pltpu.VMEM((2, CH, EMB), jnp.float32),
            pltpu.SemaphoreType.DMA,
            pltpu.SemaphoreType.DMA,
        ],
    )(_emb_sc_body)


def _emb_sc_body(tok_hbm, table_hbm, out_hbm, tok_v, rows_v, gs0, gs1):
    wid = lax.axis_index("s") * NC + lax.axis_index("c")
    chunk0 = wid * NCH

    pltpu.sync_copy(tok_hbm.at[pl.ds(chunk0, NCH)], tok_v)

    def start_gather(c, buf, sem):
        pltpu.async_copy(table_hbm.at[tok_v.at[c]], rows_v.at[buf], sem)

    def wait_gather(c, buf, sem):
        pltpu.make_async_copy(table_hbm.at[tok_v.at[c]], rows_v.at[buf], sem).wait()

    def store(c, buf):
        pltpu.sync_copy(rows_v.at[buf], out_hbm.at[pl.ds((chunk0 + c) * CH, CH)])

    start_gather(0, 0, gs0)
    start_gather(1, 1, gs1)

    @pl.loop(0, NCH // 2 - 1)
    def _(i):
        c = 2 * i
        wait_gather(c, 0, gs0)
        store(c, 0)
        start_gather(c + 2, 0, gs0)
        wait_gather(c + 1, 1, gs1)
        store(c + 1, 1)
        start_gather(c + 3, 1, gs1)

    wait_gather(NCH - 2, 0, gs0)
    store(NCH - 2, 0)
    wait_gather(NCH - 1, 1, gs1)
    store(NCH - 1, 1)


def _count_body(tok_ref, out_ref):
    t = tok_ref[...].reshape(8, 128, MAXLEN)
    out_ref[...] = jnp.sum((t != 0).astype(jnp.int32), axis=2)


_count_tc = pl.pallas_call(
    _count_body,
    out_shape=jax.ShapeDtypeStruct((8, 128), jnp.int32),
)


def kernel(tokens, table):
    tok2d = tokens.reshape(NTOK // CH, CH)
    emb_flat = _make_emb_sc()(tok2d, table)
    emb = emb_flat.reshape(B, MAXLEN, EMB)
    pro_lens = _count_tc(tokens).reshape(B)
    return emb, pro_lens


# speedup vs baseline: 16.4349x; 12.0610x over previous
import functools

import jax
import jax.numpy as jnp
from jax import lax
from jax.experimental import pallas as pl
from jax.experimental.pallas import tpu as pltpu
from jax.experimental.pallas import tpu_sc as plsc

B = 1024
MAXLEN = 1024
EMB = 128
VOCAB = 21

NC = 2
NS = 16
NW = NC * NS
NTOK = B * MAXLEN
TOK_PER_W = NTOK // NW
CH = 128
NCH = TOK_PER_W // CH

@functools.cache
def _make_emb_sc():
    mesh = plsc.VectorSubcoreMesh(
        core_axis_name="c", subcore_axis_name="s", num_cores=NC, num_subcores=NS
    )
    return functools.partial(
        pl.kernel,
        out_type=jax.ShapeDtypeStruct((NTOK, EMB), jnp.float32),
        mesh=mesh,
        scratch_types=[
            pltpu.VMEM((NCH, CH), jnp.int32),
            pltpu.VMEM((2, CH, EMB), jnp.float32),
            pltpu.VMEM_SHARED((VOCAB, EMB), jnp.float32),
            pltpu.SemaphoreType.DMA,
            pltpu.SemaphoreType.DMA,
        ],
    )(_emb_sc_body)


def _emb_sc_body(tok_hbm, table_hbm, out_hbm, tok_v, rows_v, table_v, gs0, gs1):
    wid = lax.axis_index("s") * NC + lax.axis_index("c")
    chunk0 = wid * NCH

    @pl.when(lax.axis_index("s") == 0)
    def _():
        pltpu.sync_copy(table_hbm, table_v)

    pltpu.sync_copy(tok_hbm.at[pl.ds(chunk0, NCH)], tok_v)
    plsc.subcore_barrier()

    def start_gather(c, buf, sem):
        pltpu.async_copy(table_v.at[tok_v.at[c]], rows_v.at[buf], sem)

    def wait_gather(c, buf, sem):
        pltpu.make_async_copy(table_v.at[tok_v.at[c]], rows_v.at[buf], sem).wait()

    def store(c, buf):
        pltpu.sync_copy(rows_v.at[buf], out_hbm.at[pl.ds((chunk0 + c) * CH, CH)])

    start_gather(0, 0, gs0)
    start_gather(1, 1, gs1)

    @pl.loop(0, NCH // 2 - 1)
    def _(i):
        c = 2 * i
        wait_gather(c, 0, gs0)
        store(c, 0)
        start_gather(c + 2, 0, gs0)
        wait_gather(c + 1, 1, gs1)
        store(c + 1, 1)
        start_gather(c + 3, 1, gs1)

    wait_gather(NCH - 2, 0, gs0)
    store(NCH - 2, 0)
    wait_gather(NCH - 1, 1, gs1)
    store(NCH - 1, 1)


def _count_body(tok_ref, out_ref):
    t = tok_ref[...].reshape(8, 128, MAXLEN)
    out_ref[...] = jnp.sum((t != 0).astype(jnp.int32), axis=2)


_count_tc = pl.pallas_call(
    _count_body,
    out_shape=jax.ShapeDtypeStruct((8, 128), jnp.int32),
)


def kernel(tokens, table):
    tok2d = tokens.reshape(NTOK // CH, CH)
    emb_flat = _make_emb_sc()(tok2d, table)
    emb = emb_flat.reshape(B, MAXLEN, EMB)
    pro_lens = _count_tc(tokens).reshape(B)
    return emb, pro_lens
